# baseline (speedup 1.0000x reference)
import functools

import jax
import jax.numpy as jnp
from jax import lax
from jax.experimental import pallas as pl
from jax.experimental.pallas import tpu as pltpu
from jax.experimental.pallas import tpu_sc as plsc

NUM_EMB = 1_000_000
DIM = 64
PAD = 128
B = 4096 * 200

NC = 2
NS = 16
NW = NC * NS
B_PER_W = B // NW
CHUNK = 256
N_CHUNKS = B_PER_W // CHUNK

VW_FULL = NUM_EMB // PAD
VTAIL = NUM_EMB - VW_FULL * PAD

_mesh = plsc.VectorSubcoreMesh(core_axis_name="c", subcore_axis_name="s")
_compact = pltpu.CompilerParams(use_tc_tiling_on_sc=True, needs_layout_passes=False)


@functools.partial(
    pl.kernel,
    mesh=_mesh,
    out_type=jax.ShapeDtypeStruct((NUM_EMB, PAD), jnp.float32),
    scratch_types=[
        pltpu.VMEM((DIM, PAD), jnp.float32),
        pltpu.VMEM((DIM, DIM), jnp.float32),
        pltpu.VMEM((PAD, PAD), jnp.float32),
    ],
    compiler_params=_compact,
)
def _detranspose_kernel(wt_hbm, wtail_hbm, out_hbm, buf, tbuf, obuf):
    wid = lax.axis_index("s") * NC + lax.axis_index("c")
    f16 = lax.iota(jnp.int32, 16)

    def shuffle(src, n_lanes, carry):
        def l_body(l, carry):
            lane = jnp.full((16,), l, jnp.int32)
            for c0 in range(0, DIM, 16):
                v = plsc.load_gather(src, [f16 + c0, lane])
                obuf[l, pl.ds(c0, 16)] = v
            return carry
        return lax.fori_loop(0, n_lanes, l_body, carry)

    def vw_body(vw, carry):
        pltpu.sync_copy(wt_hbm.at[:, pl.ds(vw * PAD, PAD)], buf)
        carry = shuffle(buf, PAD, carry)
        pltpu.sync_copy(obuf, out_hbm.at[pl.ds(vw * PAD, PAD), :])
        return carry

    nvw = VW_FULL // NW
    extra = VW_FULL - nvw * NW
    lax.fori_loop(0, nvw, lambda i, c: vw_body(i * NW + wid, c), 0)

    @pl.when(wid < extra)
    def _():
        vw_body(nvw * NW + wid, 0)

    @pl.when(wid == NW - 1)
    def _():
        pltpu.sync_copy(wtail_hbm, tbuf)
        shuffle(tbuf, VTAIL, 0)
        pltpu.sync_copy(obuf.at[pl.ds(0, VTAIL), :],
                        out_hbm.at[pl.ds(VW_FULL * PAD, VTAIL), :])


@functools.partial(
    pl.kernel,
    mesh=_mesh,
    out_type=jax.ShapeDtypeStruct((B, PAD), jnp.float32),
    scratch_types=[
        pltpu.VMEM((B_PER_W,), jnp.int32),
        pltpu.VMEM((CHUNK, PAD), jnp.float32),
        pltpu.VMEM((CHUNK, PAD), jnp.float32),
        pltpu.SemaphoreType.DMA,
        pltpu.SemaphoreType.DMA,
        pltpu.SemaphoreType.DMA,
        pltpu.SemaphoreType.DMA,
    ],
    compiler_params=_compact,
)
def _gather_kernel(idx_hbm, table_hbm, out_hbm, idx_v, rows0, rows1,
                   g0, g1, o0, o1):
    wid = lax.axis_index("s") * NC + lax.axis_index("c")
    base = wid * B_PER_W
    rows = (rows0, rows1)
    gsem = (g0, g1)
    osem = (o0, o1)

    def gather(i, b):
        pltpu.async_copy(
            table_hbm.at[idx_v.at[pl.ds(i * CHUNK, CHUNK)]], rows[b], gsem[b])

    def out(i, b):
        pltpu.async_copy(
            rows[b], out_hbm.at[pl.ds(base + i * CHUNK, CHUNK), :], osem[b])

    def gwait(b):
        pltpu.make_async_copy(
            table_hbm.at[idx_v.at[pl.ds(0, CHUNK)]], rows[b], gsem[b]).wait()

    def owait(b):
        pltpu.make_async_copy(
            rows[b], out_hbm.at[pl.ds(base, CHUNK), :], osem[b]).wait()

    pltpu.sync_copy(idx_hbm.at[pl.ds(base, B_PER_W)], idx_v)
    gather(0, 0)
    gather(1, 1)
    gwait(0)
    out(0, 0)

    def body(j, carry):
        i = 1 + 2 * j
        owait(0)
        gather(i + 1, 0)
        gwait(1)
        out(i, 1)
        owait(1)
        gather(i + 2, 1)
        gwait(0)
        out(i + 1, 0)
        return carry

    lax.fori_loop(0, (N_CHUNKS - 2) // 2, body, 0)

    gwait(1)
    out(N_CHUNKS - 1, 1)
    owait(0)
    owait(1)


def kernel(token_ids, weight):
    idx = token_ids.reshape(-1).astype(jnp.int32)
    wt = weight.T
    table = _detranspose_kernel(wt, wt[:, VW_FULL * PAD:])
    out = _gather_kernel(idx, table)
    return out[:, :DIM].reshape(*token_ids.shape, DIM)


# speedup vs baseline: 2.5896x; 2.5896x over previous
import functools

import jax
import jax.numpy as jnp
from jax import lax
from jax.experimental import pallas as pl
from jax.experimental.pallas import tpu as pltpu
from jax.experimental.pallas import tpu_sc as plsc

NUM_EMB = 1_000_000
DIM = 64
PAD = 128
B = 4096 * 200

NC = 2
NS = 16
NW = NC * NS
B_PER_W = B // NW
CHUNK = 512
N_CHUNKS = B_PER_W // CHUNK

_mesh = plsc.VectorSubcoreMesh(core_axis_name="c", subcore_axis_name="s")


@functools.partial(
    pl.kernel,
    mesh=_mesh,
    out_type=jax.ShapeDtypeStruct((B, PAD), jnp.float32),
    scratch_types=[
        pltpu.VMEM((B_PER_W,), jnp.int32),
        pltpu.VMEM((CHUNK, DIM), jnp.float32),
        pltpu.VMEM((CHUNK, DIM), jnp.float32),
        pltpu.SemaphoreType.DMA,
        pltpu.SemaphoreType.DMA,
        pltpu.SemaphoreType.DMA,
        pltpu.SemaphoreType.DMA,
    ],
    compiler_params=pltpu.CompilerParams(use_tc_tiling_on_sc=False),
)
def _gather_kernel(idx_hbm, table_hbm, out_hbm, idx_v, rows0, rows1,
                   g0, g1, o0, o1):
    wid = lax.axis_index("s") * NC + lax.axis_index("c")
    base = wid * B_PER_W
    rows = (rows0, rows1)
    gsem = (g0, g1)
    osem = (o0, o1)

    def gather(i, b):
        pltpu.async_copy(
            table_hbm.at[idx_v.at[pl.ds(i * CHUNK, CHUNK)]], rows[b], gsem[b])

    def out(i, b):
        pltpu.async_copy(
            rows[b],
            out_hbm.at[pl.ds(base + i * CHUNK, CHUNK), pl.ds(0, DIM)],
            osem[b])

    def gwait(b):
        pltpu.make_async_copy(
            table_hbm.at[idx_v.at[pl.ds(0, CHUNK)]], rows[b], gsem[b]).wait()

    def owait(b):
        pltpu.make_async_copy(
            rows[b],
            out_hbm.at[pl.ds(base, CHUNK), pl.ds(0, DIM)], osem[b]).wait()

    pltpu.sync_copy(idx_hbm.at[pl.ds(base, B_PER_W)], idx_v)
    gather(0, 0)
    gather(1, 1)
    gwait(0)
    out(0, 0)

    def body(j, carry):
        i = 1 + 2 * j
        owait(0)
        gather(i + 1, 0)
        gwait(1)
        out(i, 1)
        owait(1)
        gather(i + 2, 1)
        gwait(0)
        out(i + 1, 0)
        return carry

    lax.fori_loop(0, (N_CHUNKS - 2) // 2, body, 0)

    gwait(1)
    out(N_CHUNKS - 1, 1)
    owait(0)
    owait(1)


def kernel(token_ids, weight):
    idx = token_ids.reshape(-1).astype(jnp.int32)
    out = _gather_kernel(idx, weight)
    return out[:, :DIM].reshape(*token_ids.shape, DIM)
